# Initial kernel scaffold; baseline (speedup 1.0000x reference)
import jax
import jax.numpy as jnp
from jax.experimental import pallas as pl


def kernel(real, imag, edge_index, query_edges, edge_weight, W0, b0, W1, b1, lin_w, lin_b):
    raise NotImplementedError("write your pallas kernel here")


# speedup vs baseline: 9.7788x; 9.7788x over previous
import functools
import math

import jax
import jax.numpy as jnp
from jax import lax
from jax.experimental import pallas as pl
from jax.experimental.pallas import tpu as pltpu
from jax.experimental.pallas import tpu_sc as plsc

N = 10000
NP = 10240
D = 128
HID = 64
E = 320000
E2 = 2 * E
NT = 16
CH = 128
ECH = 320
E2P = NT * ECH * CH
DCH = 79
DPT = DCH * CH
EP = 32 * DPT
NQ = 10000
QCH = 3
QPW = QCH * CH
NQP = 32 * QPW
RPT = NP // NT

_mesh = lambda: plsc.VectorSubcoreMesh(core_axis_name="c", subcore_axis_name="s")
_f32 = jnp.float32
_i32 = jnp.int32


def _sc_deg_body(row_h, col_h, w_h, deg_out, deg_sp, rb, cb, ab, zb):
    c = lax.axis_index("c")
    s = lax.axis_index("s")
    wid = c * NT + s

    @pl.loop(0, RPT // 16)
    def _(i):
        zb[pl.ds(16 * i, 16)] = jnp.zeros((16,), _f32)

    pltpu.sync_copy(zb, deg_sp.at[pl.ds(s * RPT, RPT)])
    pltpu.sync_copy(row_h.at[wid], rb)
    pltpu.sync_copy(col_h.at[wid], cb)
    pltpu.sync_copy(w_h.at[wid], ab)

    @pl.loop(0, DCH)
    def _(j):
        for m in range(CH // 16):
            sl = pl.ds(16 * m, 16)
            r16 = rb[j, sl]
            c16 = cb[j, sl]
            w16 = ab[j, sl]
            ab[j, sl] = jnp.where(r16 == c16, 0.0, 0.5 * jnp.abs(w16))

    plsc.subcore_barrier()

    @pl.loop(0, DCH)
    def _(j):
        pltpu.sync_copy(ab.at[j], deg_sp.at[rb.at[j]], add=True)
        pltpu.sync_copy(ab.at[j], deg_sp.at[cb.at[j]], add=True)

    plsc.subcore_barrier()
    pltpu.sync_copy(deg_sp.at[pl.ds(s * RPT, RPT)],
                    deg_out.at[pl.ds(c * NP + s * RPT, RPT)])


def _sc_deg(row_h, col_h, w_h):
    return pl.kernel(
        _sc_deg_body,
        out_type=jax.ShapeDtypeStruct((2 * NP,), _f32),
        mesh=_mesh(),
        scratch_types=[
            pltpu.VMEM_SHARED((NP,), _f32),
            pltpu.VMEM((DCH, CH), _i32),
            pltpu.VMEM((DCH, CH), _i32),
            pltpu.VMEM((DCH, CH), _f32),
            pltpu.VMEM((RPT,), _f32),
        ],
    )(row_h, col_h, w_h)


GRP = 32
NGRP = ECH // GRP


def _mp_pass(s, coff, src_ref, dst3, src3, w2, iL, iR, raw_out, scaled_out,
             acc, dstb, srcb, wLb, wRb, rowb, dv, tmp, zr, gsem):
    @pl.loop(0, RPT // 16)
    def _(k):
        pltpu.sync_copy(zr, acc.at[pl.ds(s * RPT + 16 * k, 16)])

    plsc.subcore_barrier()

    offv = jnp.full((16,), 0, _i32) + coff

    @pl.loop(0, NGRP)
    def _(g):
        gsl = pl.ds(g * GRP, GRP)
        pltpu.sync_copy(dst3.at[s, gsl], dstb)
        pltpu.sync_copy(src3.at[s, gsl], srcb)
        pltpu.sync_copy(w2.at[iL, s, gsl], wLb)
        pltpu.sync_copy(w2.at[iR, s, gsl], wRb)

        @pl.loop(0, GRP)
        def _(j):
            for m in range(CH // 16):
                sl = pl.ds(16 * m, 16)
                srcb[j, sl] = srcb[j, sl] + offv
            pltpu.async_copy(src_ref.at[srcb.at[j]], rowb, gsem).wait()

            @pl.loop(0, CH // 16)
            def _(t):
                wL16 = wLb[j, pl.ds(t * 16, 16)]
                wR16 = wRb[j, pl.ds(t * 16, 16)]
                for l in range(16):
                    ev = t * 16 + l
                    wl = wL16[l]
                    wr = wR16[l]
                    for f in range(4):
                        sl = pl.ds(16 * f, 16)
                        rowb[ev, sl] = rowb[ev, sl] * wl
                    for f in range(4, 8):
                        sl = pl.ds(16 * f, 16)
                        rowb[ev, sl] = rowb[ev, sl] * wr

            pltpu.sync_copy(rowb, acc.at[dstb.at[j]], add=True)

    plsc.subcore_barrier()

    @pl.loop(0, RPT // 32)
    def _(k):
        base = s * RPT + 32 * k
        pltpu.sync_copy(acc.at[pl.ds(base, 32)],
                        raw_out.at[pl.ds(coff + base, 32)])
        if scaled_out is not None:
            pltpu.sync_copy(acc.at[pl.ds(base, 32)], tmp)
            for m in range(2):
                d16 = dv[pl.ds(32 * k + 16 * m, 16)]
                for l in range(16):
                    w1 = d16[l]
                    row = 16 * m + l
                    for f in range(8):
                        sl = pl.ds(16 * f, 16)
                        tmp[row, sl] = tmp[row, sl] * w1
            pltpu.sync_copy(tmp, scaled_out.at[pl.ds(coff + base, 32)])

    plsc.subcore_barrier()


def _mp_prelude(c, s, dinv2, dv, zr):
    coff = c * NP
    pltpu.sync_copy(dinv2.at[pl.ds(s * RPT, RPT)], dv)

    @pl.loop(0, 16)
    def _(i):
        for f in range(8):
            zr[i, pl.ds(16 * f, 16)] = jnp.zeros((16,), _f32)

    return coff


def _sc_layer1_body(x_stk, dst3, src3, w2, dinv2,
                    oA, oB, gA, gB, o2A, o2B,
                    acc, dstb, srcb, wLb, wRb, rowb, dv, tmp, zr, gsem):
    c = lax.axis_index("c")
    s = lax.axis_index("s")
    coff = _mp_prelude(c, s, dinv2, dv, zr)
    other = 1 - c
    sc = (acc, dstb, srcb, wLb, wRb, rowb, dv, tmp, zr, gsem)
    _mp_pass(s, coff, x_stk, dst3, src3, w2, c, c, oA, gA, *sc)
    _mp_pass(s, coff, x_stk, dst3, src3, w2, other, other, oB, gB, *sc)
    _mp_pass(s, coff, gA, dst3, src3, w2, c, c, o2A, None, *sc)
    _mp_pass(s, coff, gB, dst3, src3, w2, other, other, o2B, None, *sc)


def _sc_layer2_body(s1, dst3, src3, w2, dinv2,
                    o1, g, o2,
                    acc, dstb, srcb, wLb, wRb, rowb, dv, tmp, zr, gsem):
    c = lax.axis_index("c")
    s = lax.axis_index("s")
    coff = _mp_prelude(c, s, dinv2, dv, zr)
    other = 1 - c
    sc = (acc, dstb, srcb, wLb, wRb, rowb, dv, tmp, zr, gsem)
    _mp_pass(s, coff, s1, dst3, src3, w2, c, other, o1, g, *sc)
    _mp_pass(s, coff, g, dst3, src3, w2, c, other, o2, None, *sc)


def _mp_scratch():
    return [
        pltpu.VMEM_SHARED((NP, 128), _f32),
        pltpu.VMEM((GRP, CH), _i32),
        pltpu.VMEM((GRP, CH), _i32),
        pltpu.VMEM((GRP, CH), _f32),
        pltpu.VMEM((GRP, CH), _f32),
        pltpu.VMEM((CH, 128), _f32),
        pltpu.VMEM((RPT,), _f32),
        pltpu.VMEM((32, 128), _f32),
        pltpu.VMEM((16, 128), _f32),
        pltpu.SemaphoreType.DMA,
    ]


def _sc_layer1(x_stk, dst3, src3, w2, dinv2):
    st = jax.ShapeDtypeStruct((2 * NP, 128), _f32)
    return pl.kernel(
        _sc_layer1_body,
        out_type=[st] * 6,
        mesh=_mesh(),
        scratch_types=_mp_scratch(),
    )(x_stk, dst3, src3, w2, dinv2)


def _sc_layer2(s1, dst3, src3, w2, dinv2):
    st = jax.ShapeDtypeStruct((2 * NP, 128), _f32)
    return pl.kernel(
        _sc_layer2_body,
        out_type=[st] * 3,
        mesh=_mesh(),
        scratch_types=_mp_scratch(),
    )(s1, dst3, src3, w2, dinv2)


def _sc_qg_body(q0h, q1h, tab, out0, out1, q0b, q1b, grow, gsem):
    c = lax.axis_index("c")
    s = lax.axis_index("s")
    wid = c * NT + s
    base = wid * QPW
    pltpu.sync_copy(q0h.at[wid], q0b)
    pltpu.sync_copy(q1h.at[wid], q1b)
    for j in range(QCH):
        pltpu.async_copy(tab.at[q0b.at[j]], grow, gsem).wait()
        pltpu.sync_copy(grow, out0.at[pl.ds(base + CH * j, CH)])
        pltpu.async_copy(tab.at[q1b.at[j]], grow, gsem).wait()
        pltpu.sync_copy(grow, out1.at[pl.ds(base + CH * j, CH)])


def _sc_qg(q0h, q1h, tab):
    st = jax.ShapeDtypeStruct((NQP, 128), _f32)
    return pl.kernel(
        _sc_qg_body,
        out_type=[st] * 2,
        mesh=_mesh(),
        scratch_types=[
            pltpu.VMEM((QCH, CH), _i32),
            pltpu.VMEM((QCH, CH), _i32),
            pltpu.VMEM((CH, 128), _f32),
            pltpu.SemaphoreType.DMA,
        ],
    )(q0h, q1h, tab)


def _tc_pre_body(w_ref, r_ref, c_ref, cr_ref, ci_ref, cin_ref):
    w = jnp.where(r_ref[...] == c_ref[...], 0.0, w_ref[...])
    th = (0.5 * math.pi) * w
    hw = -0.5 * w
    ci = hw * jnp.sin(th)
    cr_ref[...] = hw * jnp.cos(th)
    ci_ref[...] = ci
    cin_ref[...] = -ci


def _tc_pre(w2, r2, c2):
    st = jax.ShapeDtypeStruct((E // 128, 128), _f32)
    return pl.pallas_call(_tc_pre_body, out_shape=[st] * 3)(w2, r2, c2)


def _tc_dinv_body(dg_ref, o_ref, o2_ref):
    dgv = dg_ref[...]
    deg = dgv[0:8] + dgv[8:16]
    dinv = jnp.where(deg > 0,
                     jax.lax.rsqrt(jnp.where(deg > 0, deg, 1.0)), 0.0)
    o_ref[...] = dinv
    o2_ref[...] = dinv * dinv


def _tc_dinv(deg16):
    st = jax.ShapeDtypeStruct((8, NP // 8), _f32)
    return pl.pallas_call(_tc_dinv_body, out_shape=[st, st])(deg16)


def _tc_scale_x_body(xr, xi, dv, o_ref):
    d = dv[...]
    o_ref[0] = xr[...] * d
    o_ref[1] = xi[...] * d


def _tc_scale_x(xrp, xip, dcol):
    blk = 1024
    grid = NP // blk
    bs = pl.BlockSpec((blk, D), lambda ii: (ii, 0))
    return pl.pallas_call(
        _tc_scale_x_body,
        grid=(grid,),
        in_specs=[bs, bs, pl.BlockSpec((blk, 1), lambda ii: (ii, 0))],
        out_specs=pl.BlockSpec((2, blk, D), lambda ii: (0, ii, 0)),
        out_shape=jax.ShapeDtypeStruct((2, NP, D), _f32),
    )(xrp, xip, dcol)


def _tc_combine1_body(xr, xi, a0, a1, a2, a3, b0, b1, b2, b3, dv, W, b,
                      pk_ref, s1_ref):
    d = dv[...]
    Wv = W[...]
    dot = lambda a, w: jax.lax.dot(a, w, preferred_element_type=_f32)
    xrv = xr[...]
    xiv = xi[...]
    t0 = d * a0[...]
    t1 = d * a1[...]
    t2 = d * a2[...]
    t3 = d * a3[...]
    u0 = 2.0 * d * b0[...] - xrv
    u1 = 2.0 * d * b1[...] - xiv
    u2 = 2.0 * d * b2[...] - xrv
    u3 = 2.0 * d * b3[...] - xiv
    xrW = dot(xrv, Wv[0])
    xiW = dot(xiv, Wv[0])
    o0 = xrW + dot(t0, Wv[1]) + dot(u0, Wv[2])
    o1 = xiW + dot(t1, Wv[1]) + dot(u1, Wv[2])
    o2 = xrW + dot(t2, Wv[1]) + dot(u2, Wv[2])
    o3 = xiW + dot(t3, Wv[1]) + dot(u3, Wv[2])
    r = o0 - o1 + b[...]
    i = o2 + o3 + b[...]
    m = (r >= 0.0).astype(_f32)
    r = r * m
    i = i * m
    pk_ref[...] = jnp.concatenate([r, i], axis=1)
    dr = d * r
    di = d * i
    s1_ref[0] = jnp.concatenate([dr, dr], axis=1)
    s1_ref[1] = jnp.concatenate([di, di], axis=1)


def _tc_combine1(xrp, xip, oA, oB, o2A, o2B, dcol, W0, b0):
    blk = 1024
    grid = NP // blk
    nb = NP // blk
    bs = pl.BlockSpec((blk, D), lambda ii: (ii, 0))
    lo = pl.BlockSpec((blk, 128), lambda ii: (ii, 0))
    hi = pl.BlockSpec((blk, 128), lambda ii: (ii + nb, 0))
    return pl.pallas_call(
        _tc_combine1_body,
        grid=(grid,),
        in_specs=[bs, bs, lo, hi, lo, hi, lo, hi, lo, hi,
                  pl.BlockSpec((blk, 1), lambda ii: (ii, 0)),
                  pl.BlockSpec((3, D, HID), lambda ii: (0, 0, 0)),
                  pl.BlockSpec((1, HID), lambda ii: (0, 0))],
        out_specs=[pl.BlockSpec((blk, 128), lambda ii: (ii, 0)),
                   pl.BlockSpec((2, blk, 128), lambda ii: (0, ii, 0))],
        out_shape=[jax.ShapeDtypeStruct((NP, 128), _f32),
                   jax.ShapeDtypeStruct((2, NP, 128), _f32)],
    )(xrp, xip, oA, oA, oB, oB, o2A, o2A, o2B, o2B, dcol, W0, b0)


def _tc_combine2_body(pk1, o1lo, o1hi, o2lo, o2hi, dv, W, b, pk_ref):
    d = dv[...]
    r1 = pk1[:, 0:HID]
    i1 = pk1[:, HID:128]
    a0 = o1lo[:, 0:HID]
    a2 = o1lo[:, HID:128]
    a1 = o1hi[:, 0:HID]
    a3 = o1hi[:, HID:128]
    b0 = o2lo[:, 0:HID]
    b2 = o2lo[:, HID:128]
    b1 = o2hi[:, 0:HID]
    b3 = o2hi[:, HID:128]
    Wv = W[...]
    dot = lambda a, w: jax.lax.dot(a, w, preferred_element_type=_f32)
    t0 = d * a0
    t1 = d * a1
    t2 = d * a2
    t3 = d * a3
    u0 = 2.0 * d * b0 - r1
    u1 = 2.0 * d * b1 - i1
    u2 = 2.0 * d * b2 - r1
    u3 = 2.0 * d * b3 - i1
    rW = dot(r1, Wv[0])
    iW = dot(i1, Wv[0])
    o0 = rW + dot(t0, Wv[1]) + dot(u0, Wv[2])
    o1 = iW + dot(t1, Wv[1]) + dot(u1, Wv[2])
    o2 = rW + dot(t2, Wv[1]) + dot(u2, Wv[2])
    o3 = iW + dot(t3, Wv[1]) + dot(u3, Wv[2])
    r = o0 - o1 + b[...]
    i = o2 + o3 + b[...]
    m = (r >= 0.0).astype(_f32)
    pk_ref[...] = jnp.concatenate([r * m, i * m], axis=1)


def _tc_combine2(pk1, o1, o2, dcol, W1, b1):
    blk = 1024
    grid = NP // blk
    nb = NP // blk
    lo = pl.BlockSpec((blk, 128), lambda ii: (ii, 0))
    hi = pl.BlockSpec((blk, 128), lambda ii: (ii + nb, 0))
    return pl.pallas_call(
        _tc_combine2_body,
        grid=(grid,),
        in_specs=[pl.BlockSpec((blk, 128), lambda ii: (ii, 0)),
                  lo, hi, lo, hi,
                  pl.BlockSpec((blk, 1), lambda ii: (ii, 0)),
                  pl.BlockSpec((3, HID, HID), lambda ii: (0, 0, 0)),
                  pl.BlockSpec((1, HID), lambda ii: (0, 0))],
        out_specs=pl.BlockSpec((blk, 128), lambda ii: (ii, 0)),
        out_shape=jax.ShapeDtypeStruct((NP, 128), _f32),
    )(pk1, o1, o1, o2, o2, dcol, W1, b1)


def _tc_final_body(ga_ref, gb_ref, lw, lb, o_ref):
    w = lw[...]
    Wac = jnp.concatenate([w[:, 0:64], w[:, 128:192]], axis=1).T
    Wbd = jnp.concatenate([w[:, 64:128], w[:, 192:256]], axis=1).T
    dot = lambda x, ww: jax.lax.dot(x, ww, preferred_element_type=_f32)
    l = dot(ga_ref[...], Wac) + dot(gb_ref[...], Wbd) + lb[...]
    m = jnp.max(l, axis=1, keepdims=True)
    lse = jnp.log(jnp.sum(jnp.exp(l - m), axis=1, keepdims=True)) + m
    o_ref[...] = l - lse


def _tc_final(ga, gb, lin_w, lin_b2):
    blk = 1024
    grid = NQP // blk
    bs = pl.BlockSpec((blk, 128), lambda ii: (ii, 0))
    return pl.pallas_call(
        _tc_final_body,
        grid=(grid,),
        in_specs=[bs, bs,
                  pl.BlockSpec((2, 256), lambda ii: (0, 0)),
                  pl.BlockSpec((1, 2), lambda ii: (0, 0))],
        out_specs=pl.BlockSpec((blk, 2), lambda ii: (ii, 0)),
        out_shape=jax.ShapeDtypeStruct((NQP, 2), _f32),
    )(ga, gb, lin_w, lin_b2)


def kernel(real, imag, edge_index, query_edges, edge_weight,
           W0, b0, W1, b1, lin_w, lin_b):
    row = edge_index[0]
    col = edge_index[1]

    shp2 = (E // 128, 128)
    cr, ci, cin = _tc_pre(edge_weight.reshape(shp2),
                          row.reshape(shp2), col.reshape(shp2))
    cr, ci, cin = (a.reshape(E) for a in (cr, ci, cin))

    padd = EP - E
    sprd = (jnp.arange(padd, dtype=_i32) % 239) + N
    row_h = jnp.concatenate([row, sprd]).reshape(32, DCH, CH)
    col_h = jnp.concatenate([col, sprd]).reshape(32, DCH, CH)
    w_h = jnp.concatenate([edge_weight, jnp.zeros((padd,), _f32)]
                          ).reshape(32, DCH, CH)
    deg_flat = _sc_deg(row_h, col_h, w_h)
    dinv, dinv2 = _tc_dinv(deg_flat.reshape(16, NP // 8))
    dinv = dinv.reshape(NP)
    dinv2 = dinv2.reshape(NP)
    dcol = dinv.reshape(NP, 1)

    pade = E2P - E2
    zpad = jnp.zeros((pade,), _f32)
    dst_s = jnp.concatenate([row, col, (jnp.arange(pade, dtype=_i32) % 239) + N])
    src_s = jnp.concatenate([col, row, jnp.arange(pade, dtype=_i32) % 251])
    cr3 = jnp.concatenate([cr, cr, zpad]).reshape(NT, ECH, CH)
    ci3 = jnp.concatenate([ci, cin, zpad]).reshape(NT, ECH, CH)
    w2 = jnp.stack([cr3, ci3])
    dst3 = dst_s.reshape(NT, ECH, CH)
    src3 = src_s.reshape(NT, ECH, CH)

    xrp = jnp.zeros((NP, D), _f32).at[0:N].set(real)
    xip = jnp.zeros((NP, D), _f32).at[0:N].set(imag)
    x_stk = _tc_scale_x(xrp, xip, dcol).reshape(2 * NP, D)
    oA, oB, gA, gB, o2A, o2B = _sc_layer1(x_stk, dst3, src3, w2, dinv2)

    pk1, s1 = _tc_combine1(xrp, xip, oA, oB, o2A, o2B, dcol,
                           W0, b0.reshape(1, HID))

    o1, g, o2 = _sc_layer2(s1.reshape(2 * NP, 128), dst3, src3, w2, dinv2)

    pk2 = _tc_combine2(pk1, o1, o2, dcol, W1, b1.reshape(1, HID))

    padq = NQP - NQ
    qsprd = jnp.arange(padq, dtype=_i32) % 251
    q0h = jnp.concatenate([query_edges[:, 0], qsprd]).reshape(32, QCH, CH)
    q1h = jnp.concatenate([query_edges[:, 1], qsprd]).reshape(32, QCH, CH)
    ga, gb = _sc_qg(q0h, q1h, pk2)

    out = _tc_final(ga, gb, lin_w, lin_b.reshape(1, 2))
    return out[:NQ]
